# baseline (speedup 1.0000x reference)
import jax
import jax.numpy as jnp
from jax.experimental import pallas as pl
from jax.experimental.pallas import tpu as pltpu

B, C_HALF, H, W = 32, 192, 64, 64
C_TOTAL = 2 * C_HALF
SUB, LANE = 32, 128
NSLOT = 16
LEAD = NSLOT - 1


def _shuffle_body(dstc_ref, dsel_ref, x1_ref, x2_ref, out1_ref, out2_ref,
                  buf_ref, in_sem, out_sem):
    def _wait_out(slot):
        pltpu.make_async_copy(
            buf_ref.at[0], out1_ref.at[:, pl.ds(0, 1)], out_sem.at[slot]
        ).wait()

    def loop_body(t, carry):
        @pl.when(t < C_TOTAL)
        def _():
            s = jax.lax.rem(t, NSLOT)

            @pl.when(t >= NSLOT)
            def _():
                _wait_out(s)

            @pl.when(t < C_HALF)
            def _():
                pltpu.make_async_copy(
                    x1_ref.at[:, pl.ds(t, 1)], buf_ref.at[s], in_sem.at[s]
                ).start()

            @pl.when(t >= C_HALF)
            def _():
                pltpu.make_async_copy(
                    x2_ref.at[:, pl.ds(t - C_HALF, 1)], buf_ref.at[s], in_sem.at[s]
                ).start()

        @pl.when(t >= LEAD)
        def _():
            u = t - LEAD
            su = jax.lax.rem(u, NSLOT)
            pltpu.make_async_copy(
                x1_ref.at[:, pl.ds(0, 1)], buf_ref.at[0], in_sem.at[su]
            ).wait()
            d = dstc_ref[u]

            @pl.when(dsel_ref[u] == 0)
            def _():
                pltpu.make_async_copy(
                    buf_ref.at[su], out1_ref.at[:, pl.ds(d, 1)], out_sem.at[su]
                ).start()

            @pl.when(dsel_ref[u] == 1)
            def _():
                pltpu.make_async_copy(
                    buf_ref.at[su], out2_ref.at[:, pl.ds(d, 1)], out_sem.at[su]
                ).start()

        return carry

    jax.lax.fori_loop(0, C_TOTAL + LEAD, loop_body, 0, unroll=False)

    def drain(s, carry):
        _wait_out(s)
        return carry

    jax.lax.fori_loop(0, NSLOT, drain, 0, unroll=False)


def kernel(x1, x2, sldj_x, fwd_idxs):
    x1r = x1.reshape(B, C_HALF, SUB, LANE)
    x2r = x2.reshape(B, C_HALF, SUB, LANE)

    dst = jnp.argsort(fwd_idxs).astype(jnp.int32)
    dsel = (dst >= C_HALF).astype(jnp.int32)
    dstc = jnp.where(dst < C_HALF, dst, dst - C_HALF).astype(jnp.int32)

    out_shape = jax.ShapeDtypeStruct((B, C_HALF, SUB, LANE), jnp.float32)
    hbm = pl.BlockSpec(memory_space=pltpu.MemorySpace.HBM)
    smem = pl.BlockSpec(memory_space=pltpu.MemorySpace.SMEM)
    out1, out2 = pl.pallas_call(
        _shuffle_body,
        in_specs=[smem, smem, hbm, hbm],
        out_specs=[hbm, hbm],
        out_shape=[out_shape, out_shape],
        scratch_shapes=[
            pltpu.VMEM((NSLOT, B, 1, SUB, LANE), jnp.float32),
            pltpu.SemaphoreType.DMA((NSLOT,)),
            pltpu.SemaphoreType.DMA((NSLOT,)),
        ],
    )(dstc, dsel, x1r, x2r)

    return (
        out1.reshape(B, C_HALF, H, W),
        out2.reshape(B, C_HALF, H, W),
        sldj_x,
    )


# speedup vs baseline: 1.1028x; 1.1028x over previous
import jax
import jax.numpy as jnp
from jax.experimental import pallas as pl
from jax.experimental.pallas import tpu as pltpu

B, C_HALF, H, W = 32, 192, 64, 64
C_TOTAL = 2 * C_HALF
SUB, LANE = 32, 128


def _permute_body(fwd_ref, x1_ref, x2_ref, o1_ref, o2_ref):
    def cp1(j, carry):
        s = fwd_ref[j]

        @pl.when(s < C_HALF)
        def _():
            o1_ref[0, pl.ds(j, 1)] = x1_ref[0, pl.ds(s, 1)]

        @pl.when(s >= C_HALF)
        def _():
            o1_ref[0, pl.ds(j, 1)] = x2_ref[0, pl.ds(s - C_HALF, 1)]

        return carry

    def cp2(j, carry):
        s = fwd_ref[j + C_HALF]

        @pl.when(s < C_HALF)
        def _():
            o2_ref[0, pl.ds(j, 1)] = x1_ref[0, pl.ds(s, 1)]

        @pl.when(s >= C_HALF)
        def _():
            o2_ref[0, pl.ds(j, 1)] = x2_ref[0, pl.ds(s - C_HALF, 1)]

        return carry

    jax.lax.fori_loop(0, C_HALF, cp1, 0, unroll=False)
    jax.lax.fori_loop(0, C_HALF, cp2, 0, unroll=False)


def kernel(x1, x2, sldj_x, fwd_idxs):
    x1r = x1.reshape(B, C_HALF, SUB, LANE)
    x2r = x2.reshape(B, C_HALF, SUB, LANE)

    block = (1, C_HALF, SUB, LANE)
    grid_spec = pltpu.PrefetchScalarGridSpec(
        num_scalar_prefetch=1,
        grid=(B,),
        in_specs=[
            pl.BlockSpec(block, lambda b, f: (b, 0, 0, 0)),
            pl.BlockSpec(block, lambda b, f: (b, 0, 0, 0)),
        ],
        out_specs=[
            pl.BlockSpec(block, lambda b, f: (b, 0, 0, 0)),
            pl.BlockSpec(block, lambda b, f: (b, 0, 0, 0)),
        ],
    )

    out_shape = jax.ShapeDtypeStruct((B, C_HALF, SUB, LANE), jnp.float32)
    out1, out2 = pl.pallas_call(
        _permute_body,
        grid_spec=grid_spec,
        out_shape=[out_shape, out_shape],
    )(fwd_idxs.astype(jnp.int32), x1r, x2r)

    return (
        out1.reshape(B, C_HALF, H, W),
        out2.reshape(B, C_HALF, H, W),
        sldj_x,
    )


# speedup vs baseline: 1.2255x; 1.1113x over previous
import jax
import jax.numpy as jnp
from jax.experimental import pallas as pl
from jax.experimental.pallas import tpu as pltpu

B, C_HALF, H, W = 32, 192, 64, 64
C_TOTAL = 2 * C_HALF
SUB, LANE = 32, 128


def _permute_body(fwd_ref, x1_ref, x2_ref, o1_ref, o2_ref):
    for j in range(C_TOTAL):
        s = fwd_ref[j]
        dst, dj = (o1_ref, j) if j < C_HALF else (o2_ref, j - C_HALF)

        @pl.when(s < C_HALF)
        def _(dst=dst, dj=dj, s=s):
            dst[0, pl.ds(dj, 1)] = x1_ref[0, pl.ds(s, 1)]

        @pl.when(s >= C_HALF)
        def _(dst=dst, dj=dj, s=s):
            dst[0, pl.ds(dj, 1)] = x2_ref[0, pl.ds(s - C_HALF, 1)]


def kernel(x1, x2, sldj_x, fwd_idxs):
    x1r = x1.reshape(B, C_HALF, SUB, LANE)
    x2r = x2.reshape(B, C_HALF, SUB, LANE)

    block = (1, C_HALF, SUB, LANE)
    grid_spec = pltpu.PrefetchScalarGridSpec(
        num_scalar_prefetch=1,
        grid=(B,),
        in_specs=[
            pl.BlockSpec(block, lambda b, f: (b, 0, 0, 0)),
            pl.BlockSpec(block, lambda b, f: (b, 0, 0, 0)),
        ],
        out_specs=[
            pl.BlockSpec(block, lambda b, f: (b, 0, 0, 0)),
            pl.BlockSpec(block, lambda b, f: (b, 0, 0, 0)),
        ],
    )

    out_shape = jax.ShapeDtypeStruct((B, C_HALF, SUB, LANE), jnp.float32)
    out1, out2 = pl.pallas_call(
        _permute_body,
        grid_spec=grid_spec,
        out_shape=[out_shape, out_shape],
    )(fwd_idxs.astype(jnp.int32), x1r, x2r)

    return (
        out1.reshape(B, C_HALF, H, W),
        out2.reshape(B, C_HALF, H, W),
        sldj_x,
    )


# speedup vs baseline: 1.2933x; 1.0553x over previous
import jax
import jax.numpy as jnp
from jax import lax
from jax.experimental import pallas as pl
from jax.experimental.pallas import tpu as pltpu
from jax.experimental.pallas import tpu_sc as plsc

B, C_HALF, H, W = 32, 192, 64, 64
C_TOTAL = 2 * C_HALF
HW = H * W

NC = 2
NS = 16
CH_PER_SUB = C_HALF // NS
PARTS = 4
BCH = B // PARTS
NCHUNK = CH_PER_SUB * PARTS


def _sc_body(fwdw_hbm, x1_hbm, x2_hbm, out1_hbm, out2_hbm,
             myfwd_v, buf, in_sem, out_sem):
    c = lax.axis_index("c")
    sid = lax.axis_index("s")
    row = (c * NS + sid) * 16
    pltpu.sync_copy(fwdw_hbm.at[pl.ds(row, 16)], myfwd_v)
    srcs = myfwd_v[...]

    def start_gather(t, slot):
        k, part = divmod(t, PARTS)
        b0 = part * BCH
        s = srcs[k]

        @pl.when(s < C_HALF)
        def _():
            pltpu.async_copy(
                x1_hbm.at[pl.ds(b0, BCH), pl.ds(s, 1)],
                buf.at[slot], in_sem.at[slot])

        @pl.when(s >= C_HALF)
        def _():
            pltpu.async_copy(
                x2_hbm.at[pl.ds(b0, BCH), pl.ds(s - C_HALF, 1)],
                buf.at[slot], in_sem.at[slot])

    def wait_in(slot):
        pltpu.make_async_copy(
            x1_hbm.at[pl.ds(0, BCH), pl.ds(0, 1)], buf.at[slot],
            in_sem.at[slot]).wait()

    def wait_out(slot, out_ref):
        pltpu.make_async_copy(
            buf.at[slot], out_ref.at[pl.ds(0, BCH), pl.ds(0, 1)],
            out_sem.at[slot]).wait()

    def start_scatter(u, slot, out_ref):
        k, part = divmod(u, PARTS)
        b0 = part * BCH
        chl = sid * CH_PER_SUB + k
        pltpu.async_copy(
            buf.at[slot], out_ref.at[pl.ds(b0, BCH), pl.ds(chl, 1)],
            out_sem.at[slot])

    def run_half(out_ref):
        start_gather(0, 0)
        for t in range(NCHUNK):
            nxt = t + 1
            slot, nslot = t % 2, nxt % 2
            if nxt < NCHUNK:
                if nxt >= 2:
                    wait_out(nslot, out_ref)
                start_gather(nxt, nslot)
            wait_in(slot)
            start_scatter(t, slot, out_ref)
        wait_out(0, out_ref)
        wait_out(1, out_ref)

    @pl.when(c == 0)
    def _():
        run_half(out1_hbm)

    @pl.when(c == 1)
    def _():
        run_half(out2_hbm)


def kernel(x1, x2, sldj_x, fwd_idxs):
    x1r = x1.reshape(B, C_HALF, HW)
    x2r = x2.reshape(B, C_HALF, HW)
    out_t = jax.ShapeDtypeStruct((B, C_HALF, HW), jnp.float32)

    fwdw = jnp.pad(
        fwd_idxs.astype(jnp.int32).reshape(NC * NS, CH_PER_SUB),
        ((0, 0), (0, 16 - CH_PER_SUB)),
    ).reshape(-1)

    f = pl.kernel(
        _sc_body,
        out_type=[out_t, out_t],
        mesh=plsc.VectorSubcoreMesh(core_axis_name="c", subcore_axis_name="s"),
        scratch_types=[
            pltpu.VMEM((16,), jnp.int32),
            pltpu.VMEM((2, BCH, 1, HW), jnp.float32),
            pltpu.SemaphoreType.DMA((2,)),
            pltpu.SemaphoreType.DMA((2,)),
        ],
    )
    out1, out2 = f(fwdw, x1r, x2r)
    return (
        out1.reshape(B, C_HALF, H, W),
        out2.reshape(B, C_HALF, H, W),
        sldj_x,
    )


# speedup vs baseline: 1.2953x; 1.0016x over previous
import jax
import jax.numpy as jnp
from jax import lax
from jax.experimental import pallas as pl
from jax.experimental.pallas import tpu as pltpu
from jax.experimental.pallas import tpu_sc as plsc

B, C_HALF, H, W = 32, 192, 64, 64
C_TOTAL = 2 * C_HALF
HW = H * W

NC = 2
NS = 16
CH_PER_SUB = C_HALF // NS
PARTS = 4
BCH = B // PARTS
NCHUNK = CH_PER_SUB * PARTS
NBUF = 3


def _sc_body(fwdw_hbm, x1_hbm, x2_hbm, out1_hbm, out2_hbm,
             myfwd_v, buf, in_sem, out_sem):
    c = lax.axis_index("c")
    sid = lax.axis_index("s")
    row = (c * NS + sid) * 16
    pltpu.sync_copy(fwdw_hbm.at[pl.ds(row, 16)], myfwd_v)
    srcs = myfwd_v[...]

    def start_gather(t, slot):
        k, part = divmod(t, PARTS)
        b0 = part * BCH
        s = srcs[k]

        @pl.when(s < C_HALF)
        def _():
            pltpu.async_copy(
                x1_hbm.at[pl.ds(b0, BCH), pl.ds(s, 1)],
                buf.at[slot], in_sem.at[slot])

        @pl.when(s >= C_HALF)
        def _():
            pltpu.async_copy(
                x2_hbm.at[pl.ds(b0, BCH), pl.ds(s - C_HALF, 1)],
                buf.at[slot], in_sem.at[slot])

    def wait_in(slot):
        pltpu.make_async_copy(
            x1_hbm.at[pl.ds(0, BCH), pl.ds(0, 1)], buf.at[slot],
            in_sem.at[slot]).wait()

    def wait_out(slot, out_ref):
        pltpu.make_async_copy(
            buf.at[slot], out_ref.at[pl.ds(0, BCH), pl.ds(0, 1)],
            out_sem.at[slot]).wait()

    def start_scatter(u, slot, out_ref):
        k, part = divmod(u, PARTS)
        b0 = part * BCH
        chl = sid * CH_PER_SUB + k
        pltpu.async_copy(
            buf.at[slot], out_ref.at[pl.ds(b0, BCH), pl.ds(chl, 1)],
            out_sem.at[slot])

    def run_half(out_ref):
        for p in range(NBUF - 1):
            start_gather(p, p)
        for t in range(NCHUNK):
            nxt = t + NBUF - 1
            if nxt < NCHUNK:
                if nxt >= NBUF:
                    wait_out(nxt % NBUF, out_ref)
                start_gather(nxt, nxt % NBUF)
            wait_in(t % NBUF)
            start_scatter(t, t % NBUF, out_ref)
        for p in range(NBUF):
            wait_out(p, out_ref)

    @pl.when(c == 0)
    def _():
        run_half(out1_hbm)

    @pl.when(c == 1)
    def _():
        run_half(out2_hbm)


def kernel(x1, x2, sldj_x, fwd_idxs):
    x1r = x1.reshape(B, C_HALF, HW)
    x2r = x2.reshape(B, C_HALF, HW)
    out_t = jax.ShapeDtypeStruct((B, C_HALF, HW), jnp.float32)

    fwdw = jnp.pad(
        fwd_idxs.astype(jnp.int32).reshape(NC * NS, CH_PER_SUB),
        ((0, 0), (0, 16 - CH_PER_SUB)),
    ).reshape(-1)

    f = pl.kernel(
        _sc_body,
        out_type=[out_t, out_t],
        mesh=plsc.VectorSubcoreMesh(core_axis_name="c", subcore_axis_name="s"),
        scratch_types=[
            pltpu.VMEM((16,), jnp.int32),
            pltpu.VMEM((NBUF, BCH, 1, HW), jnp.float32),
            pltpu.SemaphoreType.DMA((NBUF,)),
            pltpu.SemaphoreType.DMA((NBUF,)),
        ],
    )
    out1, out2 = f(fwdw, x1r, x2r)
    return (
        out1.reshape(B, C_HALF, H, W),
        out2.reshape(B, C_HALF, H, W),
        sldj_x,
    )
